# Initial kernel scaffold; baseline (speedup 1.0000x reference)
import jax
import jax.numpy as jnp
from jax.experimental import pallas as pl


def kernel(query, reference_points, value, value_spatial_shapes, W_off, b_off, W_attn, b_attn):
    raise NotImplementedError("write your pallas kernel here")


# speedup vs baseline: 3.4668x; 3.4668x over previous
import functools

import jax
import jax.numpy as jnp
from jax import lax
from jax.experimental import pallas as pl

BS = 8
LQ = 1024
EMBED = 256
HEADS = 8
HEAD_DIM = 32
LEVELS = 4
POINTS = 4
GRID_H = 32
GRID_W = 32
SUM_PTS = LEVELS * POINTS
SPATIAL = GRID_H * GRID_W
BQ = 64


def _body(q_ref, rp_ref, v_ref, wo_ref, bo_ref, wa_ref, ba_ref, out_ref):
    q = q_ref[0]
    rp = rp_ref[0, :, 0, :]
    wo = wo_ref[0]
    bo = bo_ref[0, 0]
    wa = wa_ref[0]
    ba = ba_ref[0, 0]

    off = jnp.dot(q, wo, preferred_element_type=jnp.float32) + bo[None, :]
    logits = jnp.dot(q, wa, preferred_element_type=jnp.float32) + ba[None, :]
    m = jnp.max(logits, axis=1, keepdims=True)
    e = jnp.exp(logits - m)
    attn = e / jnp.sum(e, axis=1, keepdims=True)

    ref_x = rp[:, 0:1]
    ref_y = rp[:, 1:2]
    ref_w = rp[:, 2:3]
    ref_h = rp[:, 3:4]

    iota_s = lax.broadcasted_iota(jnp.int32, (BQ, SPATIAL), 1)
    sy = iota_s >> 5
    sx = iota_s & 31

    acc = jnp.zeros((BQ, HEAD_DIM), dtype=jnp.float32)
    for level in range(LEVELS):
        s_mat = jnp.zeros((BQ, SPATIAL), dtype=jnp.float32)
        for p in range(POINTS):
            pc = level * POINTS + p
            ox = off[:, 2 * pc:2 * pc + 1]
            oy = off[:, 2 * pc + 1:2 * pc + 2]
            scale = (1.0 / POINTS) * 0.5
            gx = (ref_x + ox * scale * ref_w) * GRID_W - 0.5
            gy = (ref_y + oy * scale * ref_h) * GRID_H - 0.5
            x0f = jnp.floor(gx)
            y0f = jnp.floor(gy)
            wx1 = gx - x0f
            wx0 = 1.0 - wx1
            wy1 = gy - y0f
            wy0 = 1.0 - wy1
            x0 = jnp.clip(x0f, -2.0, 34.0).astype(jnp.int32)
            y0 = jnp.clip(y0f, -2.0, 34.0).astype(jnp.int32)
            a = attn[:, pc:pc + 1]
            ay = (jnp.where(sy == y0, wy0, 0.0)
                  + jnp.where(sy == y0 + 1, wy1, 0.0))
            ax = (jnp.where(sx == x0, wx0, 0.0)
                  + jnp.where(sx == x0 + 1, wx1, 0.0))
            s_mat = s_mat + (a * ay) * ax
        v_l = v_ref[level, 0, 0]
        acc = acc + lax.dot_general(
            s_mat, v_l, (((1,), (1,)), ((), ())),
            preferred_element_type=jnp.float32)
    out_ref[0, 0] = acc


@functools.partial(jax.jit, static_argnames=())
def kernel(query, reference_points, value, value_spatial_shapes,
           W_off, b_off, W_attn, b_attn):
    del value_spatial_shapes
    wo = W_off.reshape(EMBED, HEADS, 2 * SUM_PTS).transpose(1, 0, 2)
    wa = W_attn.reshape(EMBED, HEADS, SUM_PTS).transpose(1, 0, 2)
    bo = b_off.reshape(HEADS, 1, 2 * SUM_PTS)
    ba = b_attn.reshape(HEADS, 1, SUM_PTS)

    out = pl.pallas_call(
        _body,
        grid=(BS, HEADS, LQ // BQ),
        in_specs=[
            pl.BlockSpec((1, BQ, EMBED), lambda b, h, qb: (b, qb, 0)),
            pl.BlockSpec((1, BQ, 1, 4), lambda b, h, qb: (b, qb, 0, 0)),
            pl.BlockSpec((LEVELS, 1, 1, HEAD_DIM, SPATIAL),
                         lambda b, h, qb: (0, b, h, 0, 0)),
            pl.BlockSpec((1, EMBED, 2 * SUM_PTS), lambda b, h, qb: (h, 0, 0)),
            pl.BlockSpec((1, 1, 2 * SUM_PTS), lambda b, h, qb: (h, 0, 0)),
            pl.BlockSpec((1, EMBED, SUM_PTS), lambda b, h, qb: (h, 0, 0)),
            pl.BlockSpec((1, 1, SUM_PTS), lambda b, h, qb: (h, 0, 0)),
        ],
        out_specs=pl.BlockSpec((1, 1, BQ, HEAD_DIM),
                               lambda b, h, qb: (b, h, qb, 0)),
        out_shape=jax.ShapeDtypeStruct((BS, HEADS, LQ, HEAD_DIM),
                                       jnp.float32),
    )(query, reference_points,
      value.reshape(LEVELS, BS, HEADS, HEAD_DIM, SPATIAL),
      wo, bo, wa, ba)
    return out.transpose(0, 2, 1, 3).reshape(BS, LQ, HEADS * HEAD_DIM)


# speedup vs baseline: 120.0337x; 34.6237x over previous
import functools

import numpy as np
import jax
import jax.numpy as jnp
from jax import lax
from jax.experimental import pallas as pl
from jax.experimental.pallas import tpu as pltpu
from jax.experimental.pallas import tpu_sc as plsc

BS = 8
LQ = 1024
EMBED = 256
HEADS = 8
HEAD_DIM = 32
LEVELS = 4
POINTS = 4
GRID_H = 32
GRID_W = 32
SUM_PTS = LEVELS * POINTS
SPATIAL = GRID_H * GRID_W
NCOL = HEADS * SUM_PTS
BH = BS * HEADS

_PERM = np.array([h * 2 * SUM_PTS + p * 2 + xy
                  for xy in (0, 1) for h in range(HEADS)
                  for p in range(SUM_PTS)], dtype=np.int32)


def _corners_body(q_ref, rp_ref, wo_ref, bo_ref, wa_ref, ba_ref,
                  idx_ref, w_ref):
    q = q_ref[0]
    rpt = rp_ref[0]
    dn = (((0,), (1,)), ((), ()))
    off = lax.dot_general(wo_ref[...], q, dn,
                          preferred_element_type=jnp.float32) + bo_ref[...]
    lg = lax.dot_general(wa_ref[...], q, dn,
                         preferred_element_type=jnp.float32) + ba_ref[...]
    parts = []
    for h in range(HEADS):
        s = lg[h * SUM_PTS:(h + 1) * SUM_PTS, :]
        m = jnp.max(s, axis=0, keepdims=True)
        e = jnp.exp(s - m)
        parts.append(e / jnp.sum(e, axis=0, keepdims=True))
    attn = jnp.concatenate(parts, axis=0)

    rx = rpt[0:1, :]
    ry = rpt[1:2, :]
    rw = rpt[2:3, :]
    rh = rpt[3:4, :]
    scale = (1.0 / POINTS) * 0.5
    gx = (rx + off[:NCOL] * (scale * rw)) * GRID_W - 0.5
    gy = (ry + off[NCOL:] * (scale * rh)) * GRID_H - 0.5
    x0 = jnp.floor(gx)
    y0 = jnp.floor(gy)
    wx1 = gx - x0
    wx0 = 1.0 - wx1
    wy1 = gy - y0
    wy0 = 1.0 - wy1
    for ci, (dy, dx) in enumerate(((0, 0), (0, 1), (1, 0), (1, 1))):
        xi = x0 + dx
        yi = y0 + dy
        valid = ((xi >= 0.0) & (xi <= GRID_W - 1.0)
                 & (yi >= 0.0) & (yi <= GRID_H - 1.0))
        wxy = (wx1 if dx else wx0) * (wy1 if dy else wy0) * attn
        w_ref[0, ci] = jnp.where(valid, wxy, 0.0)
        xc = jnp.clip(xi, 0.0, GRID_W - 1.0)
        yc = jnp.clip(yi, 0.0, GRID_H - 1.0)
        idx_ref[0, ci] = (yc * GRID_W + xc).astype(jnp.int32)


def _corners(query, rp_t, wo, bo, wa, ba):
    return pl.pallas_call(
        _corners_body,
        grid=(BS,),
        in_specs=[
            pl.BlockSpec((1, LQ, EMBED), lambda b: (b, 0, 0)),
            pl.BlockSpec((1, 4, LQ), lambda b: (b, 0, 0)),
            pl.BlockSpec((EMBED, 2 * NCOL), lambda b: (0, 0)),
            pl.BlockSpec((2 * NCOL, 1), lambda b: (0, 0)),
            pl.BlockSpec((EMBED, NCOL), lambda b: (0, 0)),
            pl.BlockSpec((NCOL, 1), lambda b: (0, 0)),
        ],
        out_specs=[
            pl.BlockSpec((1, 4, NCOL, LQ), lambda b: (b, 0, 0, 0)),
            pl.BlockSpec((1, 4, NCOL, LQ), lambda b: (b, 0, 0, 0)),
        ],
        out_shape=[
            jax.ShapeDtypeStruct((BS, 4, NCOL, LQ), jnp.int32),
            jax.ShapeDtypeStruct((BS, 4, NCOL, LQ), jnp.float32),
        ],
    )(query, rp_t, wo, bo, wa, ba)


def _make_gather_kernel():
    info = plsc.get_sparse_core_info()
    nc, ns = info.num_cores, info.num_subcores
    nw = nc * ns
    bh_per = BH // nw
    nqb = LQ // 16
    mesh = plsc.VectorSubcoreMesh(core_axis_name="c", subcore_axis_name="s")

    @functools.partial(
        pl.kernel, mesh=mesh,
        compiler_params=pltpu.CompilerParams(needs_layout_passes=False),
        out_type=jax.ShapeDtypeStruct((BH, HEAD_DIM * LQ), jnp.float32),
        scratch_types=[
            pltpu.VMEM((HEAD_DIM * SPATIAL,), jnp.float32),
            pltpu.VMEM((SUM_PTS * LQ,), jnp.int32),
            pltpu.VMEM((SUM_PTS * LQ,), jnp.float32),
            pltpu.VMEM((HEAD_DIM * LQ,), jnp.float32),
        ],
    )
    def gather_kernel(value_hbm, idx_hbm, w_hbm, out_hbm,
                      table_v, idx_v, w_v, acc_v):
        wid = lax.axis_index("s") * nc + lax.axis_index("c")

        def bh_body(db, carry0):
            bh = wid * bh_per + db

            def zero_body(qb, carry1):
                zero = jnp.zeros((16,), jnp.float32)
                for c in range(HEAD_DIM):
                    acc_v[pl.ds(c * LQ + qb * 16, 16)] = zero
                return carry1

            lax.fori_loop(0, nqb, zero_body, 0)

            def lvl_body(l, carry2):
                pltpu.sync_copy(value_hbm.at[l, bh], table_v)
                pltpu.sync_copy(idx_hbm.at[bh, l], idx_v)
                pltpu.sync_copy(w_hbm.at[bh, l], w_v)

                def qb_body(qb, carry3):
                    qoff = qb * 16
                    for chalf in range(2):
                        cbase = chalf * (HEAD_DIM // 2)
                        acc = [acc_v[pl.ds((cbase + c) * LQ + qoff, 16)]
                               for c in range(HEAD_DIM // 2)]
                        for j in range(SUM_PTS):
                            sidx = idx_v[pl.ds(j * LQ + qoff, 16)]
                            wv = w_v[pl.ds(j * LQ + qoff, 16)]
                            for c in range(HEAD_DIM // 2):
                                g = plsc.load_gather(
                                    table_v,
                                    [sidx + (cbase + c) * SPATIAL])
                                acc[c] = acc[c] + wv * g
                        for c in range(HEAD_DIM // 2):
                            acc_v[pl.ds((cbase + c) * LQ + qoff, 16)] = acc[c]
                    return carry3

                lax.fori_loop(0, nqb, qb_body, 0)
                return carry2

            lax.fori_loop(0, LEVELS, lvl_body, 0)
            pltpu.sync_copy(acc_v, out_hbm.at[bh])
            return carry0

        lax.fori_loop(0, bh_per, bh_body, 0)

    return gather_kernel


_GATHER = None


def kernel(query, reference_points, value, value_spatial_shapes,
           W_off, b_off, W_attn, b_attn):
    del value_spatial_shapes
    global _GATHER
    if _GATHER is None:
        _GATHER = _make_gather_kernel()

    rp_t = reference_points[:, :, 0, :].transpose(0, 2, 1)
    wo = W_off[:, _PERM]
    bo = b_off[_PERM].reshape(2 * NCOL, 1)
    ba = b_attn.reshape(NCOL, 1)

    idx_c, w_c = _corners(query, rp_t, wo, bo, W_attn, ba)
    idx2 = (idx_c.reshape(BS, 4, HEADS, LEVELS, POINTS, LQ)
            .transpose(0, 2, 3, 4, 1, 5)
            .reshape(BH, LEVELS, SUM_PTS * LQ))
    w2 = (w_c.reshape(BS, 4, HEADS, LEVELS, POINTS, LQ)
          .transpose(0, 2, 3, 4, 1, 5)
          .reshape(BH, LEVELS, SUM_PTS * LQ))

    out = _GATHER(value.reshape(LEVELS, BH, HEAD_DIM * SPATIAL), idx2, w2)
    return (out.reshape(BS, HEADS, HEAD_DIM, LQ)
            .transpose(0, 3, 1, 2)
            .reshape(BS, LQ, HEADS * HEAD_DIM))
